# baseline (speedup 1.0000x reference)
import functools

import jax
import jax.numpy as jnp
from jax import lax
from jax.experimental import pallas as pl
from jax.experimental.pallas import tpu as pltpu
from jax.experimental.pallas import tpu_sc as plsc

N_NODES = 10000
D_IN = 256
D_OUT = 256
DH = 128
N_TILES = 16
CHUNK = 128
ROWS_PER_TILE = 624
TAIL_ROWS = N_NODES - N_TILES * ROWS_PER_TILE
DRAIN = 104


def _mm_body(x_ref, w_ref, o_ref):
    o_ref[...] = lax.dot_general(
        x_ref[...], w_ref[...],
        dimension_numbers=(((1,), (1,)), ((), ())),
        preferred_element_type=jnp.float32,
    )


def _matmul_halves(x, W):
    n = x.shape[0]
    blk = 1000
    return pl.pallas_call(
        _mm_body,
        grid=(2, n // blk),
        in_specs=[
            pl.BlockSpec((blk, D_IN), lambda c, i: (i, 0)),
            pl.BlockSpec((DH, D_IN), lambda c, i: (c, 0)),
        ],
        out_specs=pl.BlockSpec((blk, DH), lambda c, i: (c * (n // blk) + i, 0)),
        out_shape=jax.ShapeDtypeStruct((2 * n, DH), jnp.float32),
    )(x, W)


def _sc_aggregate(h2, srcs, dsts, vals, zrows, nch):
    mesh = plsc.VectorSubcoreMesh(core_axis_name="c", subcore_axis_name="s")

    @functools.partial(
        pl.kernel,
        mesh=mesh,
        out_type=jax.ShapeDtypeStruct((2, N_NODES, DH), jnp.float32),
        scratch_types=[
            pltpu.VMEM((4, CHUNK), jnp.int32),
            pltpu.VMEM((4, CHUNK), jnp.int32),
            pltpu.VMEM((4, CHUNK), jnp.float32),
            pltpu.VMEM((CHUNK, DH), jnp.float32),
            pltpu.VMEM((CHUNK, DH), jnp.float32),
            pltpu.VMEM_SHARED((N_NODES, DH), jnp.float32),
            pltpu.SemaphoreType.DMA,
            pltpu.SemaphoreType.DMA,
            pltpu.SemaphoreType.DMA,
            pltpu.SemaphoreType.DMA,
            pltpu.SemaphoreType.DMA,
            pltpu.SemaphoreType.DMA,
        ],
    )
    def body(h_ref, src_ref, dst_ref, val_ref, z_ref, out_ref,
             src_r, dst_r, val_r, rows0, rows1, acc_s,
             gsem0, gsem1, ssem0, ssem1, isem0, isem1):
        c = lax.axis_index("c")
        s = lax.axis_index("s")

        def src_off(cg):
            return ((c * N_TILES + s) * nch + cg) * CHUNK

        def edge_off(cg):
            return (s * nch + cg) * CHUNK

        def fetch_idx(cg, slot, sem):
            pltpu.async_copy(src_ref.at[pl.ds(src_off(cg), CHUNK)],
                             src_r.at[slot], sem)
            pltpu.async_copy(dst_ref.at[pl.ds(edge_off(cg), CHUNK)],
                             dst_r.at[slot], sem)
            pltpu.async_copy(val_ref.at[pl.ds(edge_off(cg), CHUNK)],
                             val_r.at[slot], sem)

        def wait_idx(slot, sem):
            pltpu.make_async_copy(src_ref.at[pl.ds(0, CHUNK)],
                                  src_r.at[slot], sem).wait()
            pltpu.make_async_copy(dst_ref.at[pl.ds(0, CHUNK)],
                                  dst_r.at[slot], sem).wait()
            pltpu.make_async_copy(val_ref.at[pl.ds(0, CHUNK)],
                                  val_r.at[slot], sem).wait()

        pltpu.sync_copy(z_ref, acc_s.at[pl.ds(s * ROWS_PER_TILE, ROWS_PER_TILE)])

        @pl.when(s == 0)
        def _():
            pltpu.sync_copy(
                z_ref.at[pl.ds(0, TAIL_ROWS)],
                acc_s.at[pl.ds(N_TILES * ROWS_PER_TILE, TAIL_ROWS)],
            )

        plsc.subcore_barrier()

        def scale(buf, m):
            def blk_body(b, carry2):
                vblk = val_r[m, pl.ds(b * 16, 16)]
                for k in range(16):
                    scal = vblk[k]
                    e = b * 16 + k
                    for f in range(DH // 16):
                        col = pl.ds(f * 16, 16)
                        buf[e, col] = buf[e, col] * scal
                return carry2

            lax.fori_loop(0, CHUNK // 16, blk_body, 0)

        fetch_idx(0, 0, isem0)
        wait_idx(0, isem0)
        fetch_idx(1, 1, isem1)
        wait_idx(1, isem1)
        fetch_idx(2, 2, isem0)
        fetch_idx(3, 3, isem1)
        pltpu.async_copy(h_ref.at[src_r.at[0]], rows0, gsem0)
        pltpu.async_copy(h_ref.at[src_r.at[1]], rows1, gsem1)

        def pair_body(p, carry):
            g = p * 2
            a0 = lax.rem(g, 4)
            a1 = lax.rem(g + 1, 4)
            a2 = lax.rem(g + 2, 4)
            a3 = lax.rem(g + 3, 4)

            pltpu.make_async_copy(h_ref.at[src_r.at[a0]], rows0, gsem0).wait()
            scale(rows0, a0)
            pltpu.async_copy(rows0, acc_s.at[dst_r.at[a0]], ssem0, add=True)

            pltpu.make_async_copy(h_ref.at[src_r.at[a1]], rows1, gsem1).wait()
            scale(rows1, a1)
            pltpu.async_copy(rows1, acc_s.at[dst_r.at[a1]], ssem1, add=True)

            pltpu.make_async_copy(rows0, acc_s.at[dst_r.at[a0]], ssem0).wait()
            wait_idx(a2, isem0)
            pltpu.async_copy(h_ref.at[src_r.at[a2]], rows0, gsem0)
            fetch_idx(lax.rem(g + 4, nch), a0, isem0)

            pltpu.make_async_copy(rows1, acc_s.at[dst_r.at[a1]], ssem1).wait()
            wait_idx(a3, isem1)
            pltpu.async_copy(h_ref.at[src_r.at[a3]], rows1, gsem1)
            fetch_idx(lax.rem(g + 5, nch), a1, isem1)
            return carry

        lax.fori_loop(0, nch // 2, pair_body, 0)
        pltpu.make_async_copy(h_ref.at[src_r.at[0]], rows0, gsem0).wait()
        pltpu.make_async_copy(h_ref.at[src_r.at[1]], rows1, gsem1).wait()
        wait_idx(0, isem0)
        wait_idx(1, isem1)
        plsc.subcore_barrier()

        def drain_chunk(row0, nrows):
            sl = pl.ds(row0, nrows)
            pltpu.sync_copy(acc_s.at[sl], rows0.at[pl.ds(0, nrows)])

            def relu_body(i, carry2):
                for f in range(DH // 16):
                    col = pl.ds(f * 16, 16)
                    rows0[i, col] = jnp.maximum(rows0[i, col], 0.0)
                return carry2

            lax.fori_loop(0, nrows, relu_body, 0)
            pltpu.sync_copy(rows0.at[pl.ds(0, nrows)], out_ref.at[c, sl])

        base = s * ROWS_PER_TILE
        for k in range(ROWS_PER_TILE // DRAIN):
            drain_chunk(base + k * DRAIN, DRAIN)

        @pl.when(s == 0)
        def _():
            drain_chunk(N_TILES * ROWS_PER_TILE, TAIL_ROWS)

    return body(h2, srcs, dsts, vals, zrows)


def kernel(x, W, adj_values, edge_index):
    n, e = x.shape[0], adj_values.shape[0]
    nch = -(-e // (N_TILES * CHUNK))
    nch += nch % 2
    e_pad = nch * N_TILES * CHUNK
    pad = e_pad - e

    h2 = _matmul_halves(x, W)

    src = jnp.pad(edge_index[1], (0, pad))
    srcs = jnp.concatenate([src, src + n])
    dsts = jnp.pad(edge_index[0], (0, pad))
    vals = jnp.pad(adj_values, (0, pad))
    zrows = jnp.zeros((ROWS_PER_TILE, DH), jnp.float32)

    out2 = _sc_aggregate(h2, srcs, dsts, vals, zrows, nch)
    return out2.transpose(1, 0, 2).reshape(n, D_OUT)


# speedup vs baseline: 1.0736x; 1.0736x over previous
import functools

import jax
import jax.numpy as jnp
import numpy as np
from jax import lax
from jax.experimental import pallas as pl
from jax.experimental.pallas import tpu as pltpu
from jax.experimental.pallas import tpu_sc as plsc

N_NODES = 10000
D_IN = 256
D_OUT = 256
DH = 128
N_TILES = 16
CHUNK = 128
ROWS_PER_TILE = 624
TAIL_ROWS = N_NODES - N_TILES * ROWS_PER_TILE
DRAIN = 104

_Q = np.empty(DH, np.int32)
_Q[0::2] = np.arange(64, dtype=np.int32)
_Q[1::2] = np.arange(64, 128, dtype=np.int32)
_W_PERM = np.concatenate([_Q, DH + _Q])


def _mm_body(x_ref, w_ref, o_ref):
    o_ref[...] = lax.dot_general(
        x_ref[...], w_ref[...],
        dimension_numbers=(((1,), (1,)), ((), ())),
        preferred_element_type=jnp.float32,
    ).astype(jnp.bfloat16)


def _matmul_halves(x, W):
    n = x.shape[0]
    blk = 1000
    return pl.pallas_call(
        _mm_body,
        grid=(2, n // blk),
        in_specs=[
            pl.BlockSpec((blk, D_IN), lambda c, i: (i, 0)),
            pl.BlockSpec((DH, D_IN), lambda c, i: (c, 0)),
        ],
        out_specs=pl.BlockSpec((blk, DH), lambda c, i: (c * (n // blk) + i, 0)),
        out_shape=jax.ShapeDtypeStruct((2 * n, DH), jnp.bfloat16),
    )(x, W)


def _sc_aggregate(h2, srcs, dsts, vals, zrows, nch):
    mesh = plsc.VectorSubcoreMesh(core_axis_name="c", subcore_axis_name="s")

    @functools.partial(
        pl.kernel,
        mesh=mesh,
        compiler_params=pltpu.CompilerParams(
            needs_layout_passes=False, use_tc_tiling_on_sc=False),
        out_type=jax.ShapeDtypeStruct((2, N_NODES, DH), jnp.float32),
        scratch_types=[
            pltpu.VMEM((4, CHUNK), jnp.int32),
            pltpu.VMEM((4, CHUNK), jnp.int32),
            pltpu.VMEM((4, CHUNK), jnp.float32),
            pltpu.VMEM((CHUNK, DH // 2), jnp.int32),
            pltpu.VMEM((CHUNK, DH // 2), jnp.int32),
            pltpu.VMEM((CHUNK, DH), jnp.float32),
            pltpu.VMEM((CHUNK, DH), jnp.float32),
            pltpu.VMEM_SHARED((N_NODES, DH), jnp.float32),
            pltpu.SemaphoreType.DMA,
            pltpu.SemaphoreType.DMA,
            pltpu.SemaphoreType.DMA,
            pltpu.SemaphoreType.DMA,
            pltpu.SemaphoreType.DMA,
            pltpu.SemaphoreType.DMA,
        ],
    )
    def body(h_ref, src_ref, dst_ref, val_ref, z_ref, out_ref,
             src_r, dst_r, val_r, gb0, gb1, sb0, sb1, acc_s,
             gsem0, gsem1, ssem0, ssem1, isem0, isem1):
        c = lax.axis_index("c")
        s = lax.axis_index("s")

        def src_off(cg):
            return ((c * N_TILES + s) * nch + cg) * CHUNK

        def edge_off(cg):
            return (s * nch + cg) * CHUNK

        def fetch_idx(cg, slot, sem):
            pltpu.async_copy(src_ref.at[pl.ds(src_off(cg), CHUNK)],
                             src_r.at[slot], sem)
            pltpu.async_copy(dst_ref.at[pl.ds(edge_off(cg), CHUNK)],
                             dst_r.at[slot], sem)
            pltpu.async_copy(val_ref.at[pl.ds(edge_off(cg), CHUNK)],
                             val_r.at[slot], sem)

        def wait_idx(slot, sem):
            pltpu.make_async_copy(src_ref.at[pl.ds(0, CHUNK)],
                                  src_r.at[slot], sem).wait()
            pltpu.make_async_copy(dst_ref.at[pl.ds(0, CHUNK)],
                                  dst_r.at[slot], sem).wait()
            pltpu.make_async_copy(val_ref.at[pl.ds(0, CHUNK)],
                                  val_r.at[slot], sem).wait()

        pltpu.sync_copy(z_ref, acc_s.at[pl.ds(s * ROWS_PER_TILE, ROWS_PER_TILE)])

        @pl.when(s == 0)
        def _():
            pltpu.sync_copy(
                z_ref.at[pl.ds(0, TAIL_ROWS)],
                acc_s.at[pl.ds(N_TILES * ROWS_PER_TILE, TAIL_ROWS)],
            )

        plsc.subcore_barrier()

        def scale(gb, sb, m):
            @plsc.parallel_loop(0, CHUNK // 16)
            def blk_body(b):
                vblk = val_r[m, pl.ds(b * 16, 16)]
                for k in range(16):
                    scal = vblk[k]
                    e = b * 16 + k
                    for t in range(DH // 32):
                        w = gb[e, pl.ds(t * 16, 16)]
                        lo = plsc.bitcast(
                            lax.shift_left(w, 16), jnp.float32)
                        hi = plsc.bitcast(
                            jnp.bitwise_and(w, jnp.int32(-65536)),
                            jnp.float32)
                        sb[e, pl.ds(t * 16, 16)] = lo * scal
                        sb[e, pl.ds(64 + t * 16, 16)] = hi * scal

        fetch_idx(0, 0, isem0)
        wait_idx(0, isem0)
        fetch_idx(1, 1, isem1)
        wait_idx(1, isem1)
        fetch_idx(2, 2, isem0)
        fetch_idx(3, 3, isem1)
        pltpu.async_copy(h_ref.at[src_r.at[0]], gb0, gsem0)
        pltpu.async_copy(h_ref.at[src_r.at[1]], gb1, gsem1)

        def pair_body(p, carry):
            g = p * 2
            a0 = lax.rem(g, 4)
            a1 = lax.rem(g + 1, 4)
            a2 = lax.rem(g + 2, 4)
            a3 = lax.rem(g + 3, 4)

            pltpu.make_async_copy(h_ref.at[src_r.at[a0]], gb0, gsem0).wait()
            scale(gb0, sb0, a0)
            pltpu.async_copy(sb0, acc_s.at[dst_r.at[a0]], ssem0, add=True)
            wait_idx(a2, isem0)
            pltpu.async_copy(h_ref.at[src_r.at[a2]], gb0, gsem0)

            pltpu.make_async_copy(h_ref.at[src_r.at[a1]], gb1, gsem1).wait()
            scale(gb1, sb1, a1)
            pltpu.async_copy(sb1, acc_s.at[dst_r.at[a1]], ssem1, add=True)
            wait_idx(a3, isem1)
            pltpu.async_copy(h_ref.at[src_r.at[a3]], gb1, gsem1)

            pltpu.make_async_copy(sb0, acc_s.at[dst_r.at[a0]], ssem0).wait()
            fetch_idx(lax.rem(g + 4, nch), a0, isem0)
            pltpu.make_async_copy(sb1, acc_s.at[dst_r.at[a1]], ssem1).wait()
            fetch_idx(lax.rem(g + 5, nch), a1, isem1)
            return carry

        lax.fori_loop(0, nch // 2, pair_body, 0)
        pltpu.make_async_copy(h_ref.at[src_r.at[0]], gb0, gsem0).wait()
        pltpu.make_async_copy(h_ref.at[src_r.at[1]], gb1, gsem1).wait()
        wait_idx(0, isem0)
        wait_idx(1, isem1)
        plsc.subcore_barrier()

        def drain_chunk(row0, nrows):
            sl = pl.ds(row0, nrows)
            pltpu.sync_copy(acc_s.at[sl], sb0.at[pl.ds(0, nrows)])

            @plsc.parallel_loop(0, nrows)
            def relu_body(i):
                for f in range(DH // 16):
                    col = pl.ds(f * 16, 16)
                    sb0[i, col] = jnp.maximum(sb0[i, col], 0.0)
            pltpu.sync_copy(sb0.at[pl.ds(0, nrows)], out_ref.at[c, sl])

        base = s * ROWS_PER_TILE
        for k in range(ROWS_PER_TILE // DRAIN):
            drain_chunk(base + k * DRAIN, DRAIN)

        @pl.when(s == 0)
        def _():
            drain_chunk(N_TILES * ROWS_PER_TILE, TAIL_ROWS)

    return body(h2, srcs, dsts, vals, zrows)


def kernel(x, W, adj_values, edge_index):
    n, e = x.shape[0], adj_values.shape[0]
    nch = -(-e // (N_TILES * CHUNK))
    nch += nch % 2
    e_pad = nch * N_TILES * CHUNK
    pad = e_pad - e

    h2 = _matmul_halves(x, W[jnp.asarray(_W_PERM)])
    h2 = lax.bitcast_convert_type(
        h2.reshape(2 * n, DH // 2, 2), jnp.int32)

    src = jnp.pad(edge_index[1], (0, pad))
    srcs = jnp.concatenate([src, src + n])
    dsts = jnp.pad(edge_index[0], (0, pad))
    vals = jnp.pad(adj_values, (0, pad))
    zrows = jnp.zeros((ROWS_PER_TILE, DH), jnp.float32)

    out2 = _sc_aggregate(h2, srcs, dsts, vals, zrows, nch)
    return out2.transpose(1, 0, 2).reshape(n, D_OUT)
